# baseline (speedup 1.0000x reference)
import functools

import jax
import jax.numpy as jnp
from jax.experimental import pallas as pl
from jax.experimental.pallas import tpu as pltpu

N = 10000
D = 128
BM = 400


def _fused_kernel(adj_ref, x_ref, w_ref, b_ref, o_ref, s_ref):
    @pl.when(pl.program_id(0) == 0)
    def _compute_support():
        s_ref[...] = jnp.dot(x_ref[...], w_ref[...],
                             preferred_element_type=jnp.float32)

    last = pl.num_programs(0) - 1

    @pl.when(pl.program_id(0) != last)
    def _full_dot():
        o_ref[...] = jnp.dot(adj_ref[...], s_ref[...],
                             preferred_element_type=jnp.float32) + b_ref[...]

    @pl.when(pl.program_id(0) == last)
    def _fast_tail_dot():
        o_ref[...] = jnp.dot(adj_ref[...].astype(jnp.bfloat16),
                             s_ref[...].astype(jnp.bfloat16),
                             preferred_element_type=jnp.float32) + b_ref[...]


@functools.partial(jax.jit, static_argnames=())
def kernel(x, adj, W, b):
    b2d = b.reshape(1, D)
    out = pl.pallas_call(
        _fused_kernel,
        grid=(N // BM,),
        in_specs=[
            pl.BlockSpec((BM, N), lambda m: (m, 0)),
            pl.BlockSpec((N, D), lambda m: (0, 0)),
            pl.BlockSpec((D, D), lambda m: (0, 0)),
            pl.BlockSpec((1, D), lambda m: (0, 0)),
        ],
        out_specs=pl.BlockSpec((BM, D), lambda m: (m, 0)),
        out_shape=jax.ShapeDtypeStruct((N, D), jnp.float32),
        scratch_shapes=[pltpu.VMEM((N, D), jnp.float32)],
        compiler_params=pltpu.CompilerParams(
            dimension_semantics=("arbitrary",)),
    )(adj, x, W, b2d)
    return out


# speedup vs baseline: 1.0048x; 1.0048x over previous
import jax
import jax.numpy as jnp
from jax.experimental import pallas as pl
from jax.experimental.pallas import tpu as pltpu

N = 10000
D = 128
BM = 400


def _fused_kernel(adj_ref, x_ref, w_ref, b_ref, o_ref, s_ref):
    @pl.when(pl.program_id(0) == 0)
    def _compute_support():
        s_ref[...] = jnp.dot(x_ref[...], w_ref[...],
                             preferred_element_type=jnp.float32)

    o_ref[...] = jnp.dot(adj_ref[...], s_ref[...],
                         preferred_element_type=jnp.float32) + b_ref[...]


@jax.jit
def kernel(x, adj, W, b):
    b2d = b.reshape(1, D)
    out = pl.pallas_call(
        _fused_kernel,
        grid=(N // BM,),
        in_specs=[
            pl.BlockSpec((BM, N), lambda m: (m, 0)),
            pl.BlockSpec((N, D), lambda m: (0, 0)),
            pl.BlockSpec((D, D), lambda m: (0, 0)),
            pl.BlockSpec((1, D), lambda m: (0, 0)),
        ],
        out_specs=pl.BlockSpec((BM, D), lambda m: (m, 0)),
        out_shape=jax.ShapeDtypeStruct((N, D), jnp.float32),
        scratch_shapes=[pltpu.VMEM((N, D), jnp.float32)],
        compiler_params=pltpu.CompilerParams(
            dimension_semantics=("arbitrary",)),
    )(adj, x, W, b2d)
    return out
